# baseline (speedup 1.0000x reference)
import functools

import jax
import jax.numpy as jnp
from jax import lax
from jax.experimental import pallas as pl
from jax.experimental.pallas import tpu as pltpu
from jax.experimental.pallas import tpu_sc as plsc

N = 10000
E = 320000
D = 128

NC = 2
NS = 16
NW = NC * NS
L = 16

BLK = 128
BPW = 80
NBLKS = BPW * NW
EPAD = NBLKS * BLK
CHUNK = EPAD // NW

NPAD = 10240
SENT = NPAD - 1
SLC = NPAD // NS

@functools.cache
def _mesh():
    return plsc.VectorSubcoreMesh(
        core_axis_name="c", subcore_axis_name="s", num_cores=NC, num_subcores=NS
    )


def _fill_f32(ref, nrows, value):
    vec = jnp.full((L,), value, jnp.float32)

    def body(j, _):
        for i in range(BLK // L):
            ref[j, pl.ds(i * L, L)] = vec
        return 0

    lax.fori_loop(0, nrows, body, 0)


def _fill_1d_f32(ref, n, value):
    vec = jnp.full((L,), value, jnp.float32)

    def body(i, _):
        ref[pl.ds(i * L, L)] = vec
        return 0

    lax.fori_loop(0, n // L, body, 0)


def _hist_body(src_hbm, dst_hbm, out_hbm, idx_s, idx_d, loc, rbuf, sh_s, sh_d):
    cid = lax.axis_index("c")
    sid = lax.axis_index("s")
    wid = cid * NS + sid


    start = wid * CHUNK
    pltpu.sync_copy(src_hbm.at[pl.ds(start, CHUNK)], idx_s)
    pltpu.sync_copy(dst_hbm.at[pl.ds(start, CHUNK)], idx_d)

    def hist_one(idx_ref, sh_grid, which):
        def run():
            _fill_1d_f32(loc, NPAD, 0.0)
            ones16 = jnp.full((L,), 1.0, jnp.float32)

            def upd(i, _):
                idx16 = idx_ref[pl.ds(i * L, L)]
                plsc.addupdate_scatter(loc, [idx16], ones16)
                return 0

            lax.fori_loop(0, CHUNK // L, upd, 0)
            pltpu.sync_copy(loc, sh_grid.at[sid])
            plsc.subcore_barrier()
            pltpu.sync_copy(sh_grid.at[:, pl.ds(sid * SLC, SLC)], rbuf)
            acc = loc.at[pl.ds(0, SLC)]

            def red(i, _):
                s = rbuf[0, pl.ds(i * L, L)]
                for r in range(1, NS):
                    s = s + rbuf[r, pl.ds(i * L, L)]
                acc[pl.ds(i * L, L)] = s
                return 0

            lax.fori_loop(0, SLC // L, red, 0)
            pltpu.sync_copy(acc, out_hbm.at[cid, which, pl.ds(sid * SLC, SLC)])

        run()

    hist_one(idx_s, sh_s, 0)
    hist_one(idx_d, sh_d, 1)


@functools.cache
def _hist_kernel():
    return pl.kernel(
        _hist_body,
        out_type=jax.ShapeDtypeStruct((NC, 2, NPAD), jnp.float32),
        mesh=_mesh(),
        compiler_params=pltpu.CompilerParams(needs_layout_passes=False),
        scratch_types=[
            pltpu.VMEM((NPAD,), jnp.int32),
            pltpu.VMEM((NPAD,), jnp.int32),
            pltpu.VMEM((NPAD,), jnp.float32),
            pltpu.VMEM((NS, SLC), jnp.float32),
            pltpu.VMEM_SHARED((NS, NPAD), jnp.float32),
            pltpu.VMEM_SHARED((NS, NPAD), jnp.float32),
        ],
    )


def _cscatter_body(src_hbm, dst_hbm, nin_hbm, out_hbm,
                   idx_s, idx_d, vals, stage, sh_nin, sh_c):
    cid = lax.axis_index("c")
    sid = lax.axis_index("s")
    wid = cid * NS + sid

    _fill_1d_f32(stage, SLC, 0.0)
    pltpu.sync_copy(stage, sh_c.at[pl.ds(sid * SLC, SLC)])
    pltpu.sync_copy(nin_hbm.at[pl.ds(sid * SLC, SLC)], stage)
    pltpu.sync_copy(stage, sh_nin.at[pl.ds(sid * SLC, SLC)])

    start = wid * CHUNK
    pltpu.sync_copy(src_hbm.at[pl.ds(start, CHUNK)], idx_s)
    pltpu.sync_copy(dst_hbm.at[pl.ds(start, CHUNK)], idx_d)

    plsc.subcore_barrier()

    pltpu.sync_copy(sh_nin.at[idx_d], vals)
    pltpu.sync_copy(vals, sh_c.at[idx_s], add=True)

    plsc.subcore_barrier()

    pltpu.sync_copy(sh_c.at[pl.ds(sid * SLC, SLC)],
                    out_hbm.at[cid, pl.ds(sid * SLC, SLC)])


@functools.cache
def _cscatter_kernel():
    return pl.kernel(
        _cscatter_body,
        out_type=jax.ShapeDtypeStruct((NC, NPAD), jnp.float32),
        mesh=_mesh(),
        scratch_types=[
            pltpu.VMEM((CHUNK,), jnp.int32),
            pltpu.VMEM((CHUNK,), jnp.int32),
            pltpu.VMEM((CHUNK,), jnp.float32),
            pltpu.VMEM((SLC,), jnp.float32),
            pltpu.VMEM_SHARED((NPAD,), jnp.float32),
            pltpu.VMEM_SHARED((NPAD,), jnp.float32),
        ],
    )


def _norm_body(deg_ref, nout_ref, nin_ref):
    x = deg_ref[...]
    deg_s = x[0, 0] + x[1, 0]
    deg_d = x[0, 1] + x[1, 1]
    nout_ref[...] = lax.rsqrt(jnp.maximum(deg_s, 1.0))
    nin_ref[...] = lax.rsqrt(jnp.maximum(deg_d, 1.0))


_ROWS = NPAD // 128


def _norms(deg_part):
    return pl.pallas_call(
        _norm_body,
        out_shape=[jax.ShapeDtypeStruct((_ROWS, 128), jnp.float32)] * 2,
    )(deg_part)


def _final_body(c0_ref, c1_ref, no_ref, f_ref, w2_ref, b2_ref, out_ref):
    j = pl.program_id(0)
    w = (c0_ref[...] + c1_ref[...]) * no_ref[...] * jnp.float32(1.0 / N)
    part = jnp.dot(w[0], f_ref[...], preferred_element_type=jnp.float32)

    @pl.when(j == 0)
    def _():
        out_ref[...] = jnp.zeros_like(out_ref)

    out_ref[...] += part

    @pl.when(j == _ROWS - 1)
    def _():
        out_ref[...] = (
            jnp.dot(out_ref[...], w2_ref[...], preferred_element_type=jnp.float32)
            + b2_ref[...]
        )


def _final(c0, c1, norm_out, f_pad, W2, b2):
    return pl.pallas_call(
        _final_body,
        grid=(_ROWS,),
        in_specs=[
            pl.BlockSpec((1, 1, 128), lambda j: (j, 0, 0)),
            pl.BlockSpec((1, 1, 128), lambda j: (j, 0, 0)),
            pl.BlockSpec((1, 1, 128), lambda j: (j, 0, 0)),
            pl.BlockSpec((128, 128), lambda j: (j, 0)),
            pl.BlockSpec((128, 128), lambda j: (0, 0)),
            pl.BlockSpec((1, 128), lambda j: (0, 0)),
        ],
        out_specs=pl.BlockSpec((1, 128), lambda j: (0, 0)),
        out_shape=jax.ShapeDtypeStruct((1, D), jnp.float32),
    )(c0, c1, norm_out, f_pad, W2, b2)


def kernel(feature, edge_index, W1, b1, W2, b2):
    pad = jnp.full((2, EPAD - E), SENT, jnp.int32)
    ei = jnp.concatenate([edge_index, pad], axis=1)
    src2 = ei[0]
    dst2 = ei[1]

    deg_part = _hist_kernel()(src2, dst2)
    norm_out, norm_in = _norms(deg_part.reshape(NC, 2, _ROWS, 128))
    c_part = _cscatter_kernel()(src2, dst2, norm_in.reshape(NPAD))

    f_pad = jnp.pad(feature, ((0, NPAD - N), (0, 0)))
    out = _final(
        c_part[0].reshape(_ROWS, 1, 128),
        c_part[1].reshape(_ROWS, 1, 128),
        norm_out.reshape(_ROWS, 1, 128),
        f_pad,
        W2,
        b2.reshape(1, D),
    )
    return out


# speedup vs baseline: 1.0506x; 1.0506x over previous
import functools

import jax
import jax.numpy as jnp
from jax import lax
from jax.experimental import pallas as pl
from jax.experimental.pallas import tpu as pltpu
from jax.experimental.pallas import tpu_sc as plsc

N = 10000
E = 320000
D = 128

NC = 2
NS = 16
L = 16
U = 8

EPAD = 327680
C2 = EPAD // NS
HALF = C2 // NC

NPAD = 10240
SENT = NPAD - 1
SLC = NPAD // NS


@functools.cache
def _mesh():
    return plsc.VectorSubcoreMesh(
        core_axis_name="c", subcore_axis_name="s", num_cores=NC, num_subcores=NS
    )


def _zero_fill(ref, n):
    z = jnp.zeros((L,), jnp.float32)

    def body(o, _):
        for k in range(U):
            ref[pl.ds((o * U + k) * L, L)] = z
        return 0

    lax.fori_loop(0, n // (L * U), body, 0)


def _rsqrt16(x):
    i = plsc.bitcast(x, jnp.int32)
    i = jnp.int32(0x5F3759DF) - (i >> 1)
    y = plsc.bitcast(i, jnp.float32)
    for _ in range(3):
        y = y * (jnp.float32(1.5) - jnp.float32(0.5) * x * y * y)
    return y


def _mega_body(src_hbm, dst_hbm, out_hbm,
               idx_s, idx_d, loc, rbuf, nin_loc, nout_s, wstage,
               grid_s, grid_d, sh_nin):
    cid = lax.axis_index("c")
    sid = lax.axis_index("s")

    pltpu.sync_copy(src_hbm.at[pl.ds(sid * C2, C2)], idx_s)
    pltpu.sync_copy(dst_hbm.at[pl.ds(sid * C2, C2)], idx_d)

    ones16 = jnp.full((L,), 1.0, jnp.float32)

    def hist(idx_ref, grid):
        _zero_fill(loc, NPAD)

        def upd(o, _):
            for k in range(U):
                i = o * U + k
                idx16 = idx_ref[pl.ds(i * L, L)]
                plsc.addupdate_scatter(loc, [idx16], ones16)
            return 0

        lax.fori_loop(0, C2 // (L * U), upd, 0)
        pltpu.sync_copy(loc, grid.at[sid])

    hist(idx_s, grid_s)
    hist(idx_d, grid_d)
    plsc.subcore_barrier()

    def reduce_norm(grid, dst_buf):
        pltpu.sync_copy(grid.at[:, pl.ds(sid * SLC, SLC)], rbuf)

        def red(i, _):
            s = rbuf[0, pl.ds(i * L, L)]
            for r in range(1, NS):
                s = s + rbuf[r, pl.ds(i * L, L)]
            dst_buf[pl.ds(i * L, L)] = _rsqrt16(jnp.maximum(s, 1.0))
            return 0

        lax.fori_loop(0, SLC // L, red, 0)

    reduce_norm(grid_s, nout_s)
    reduce_norm(grid_d, wstage)
    pltpu.sync_copy(wstage, sh_nin.at[pl.ds(sid * SLC, SLC)])
    plsc.subcore_barrier()

    pltpu.sync_copy(sh_nin, nin_loc)

    _zero_fill(loc, NPAD)
    base = cid * HALF

    def upd2(o, _):
        for k in range(U):
            i = o * U + k
            d16 = idx_d[pl.ds(base + i * L, L)]
            v16 = plsc.load_gather(nin_loc, [d16])
            s16 = idx_s[pl.ds(base + i * L, L)]
            plsc.addupdate_scatter(loc, [s16], v16)
        return 0

    lax.fori_loop(0, HALF // (L * U), upd2, 0)
    pltpu.sync_copy(loc, grid_s.at[sid])
    plsc.subcore_barrier()

    pltpu.sync_copy(grid_s.at[:, pl.ds(sid * SLC, SLC)], rbuf)
    inv_n = jnp.float32(1.0 / N)

    def red2(i, _):
        s = rbuf[0, pl.ds(i * L, L)]
        for r in range(1, NS):
            s = s + rbuf[r, pl.ds(i * L, L)]
        wstage[pl.ds(i * L, L)] = s * nout_s[pl.ds(i * L, L)] * inv_n
        return 0

    lax.fori_loop(0, SLC // L, red2, 0)
    pltpu.sync_copy(wstage, out_hbm.at[cid, pl.ds(sid * SLC, SLC)])


@functools.cache
def _mega_kernel():
    return pl.kernel(
        _mega_body,
        out_type=jax.ShapeDtypeStruct((NC, NPAD), jnp.float32),
        mesh=_mesh(),
        compiler_params=pltpu.CompilerParams(needs_layout_passes=False),
        scratch_types=[
            pltpu.VMEM((C2,), jnp.int32),
            pltpu.VMEM((C2,), jnp.int32),
            pltpu.VMEM((NPAD,), jnp.float32),
            pltpu.VMEM((NS, SLC), jnp.float32),
            pltpu.VMEM((NPAD,), jnp.float32),
            pltpu.VMEM((SLC,), jnp.float32),
            pltpu.VMEM((SLC,), jnp.float32),
            pltpu.VMEM_SHARED((NS, NPAD), jnp.float32),
            pltpu.VMEM_SHARED((NS, NPAD), jnp.float32),
            pltpu.VMEM_SHARED((NPAD,), jnp.float32),
        ],
    )


_ROWS = NPAD // 128


def _final_body(w0_ref, w1_ref, f_ref, w2_ref, b2_ref, out_ref):
    j = pl.program_id(0)
    w = w0_ref[...] + w1_ref[...]
    part = jnp.dot(w[0], f_ref[...], preferred_element_type=jnp.float32)

    @pl.when(j == 0)
    def _():
        out_ref[...] = jnp.zeros_like(out_ref)

    out_ref[...] += part

    @pl.when(j == _ROWS - 1)
    def _():
        out_ref[...] = (
            jnp.dot(out_ref[...], w2_ref[...], preferred_element_type=jnp.float32)
            + b2_ref[...]
        )


def _final(w0, w1, f_pad, W2, b2):
    return pl.pallas_call(
        _final_body,
        grid=(_ROWS,),
        in_specs=[
            pl.BlockSpec((1, 1, 128), lambda j: (j, 0, 0)),
            pl.BlockSpec((1, 1, 128), lambda j: (j, 0, 0)),
            pl.BlockSpec((128, 128), lambda j: (j, 0)),
            pl.BlockSpec((128, 128), lambda j: (0, 0)),
            pl.BlockSpec((1, 128), lambda j: (0, 0)),
        ],
        out_specs=pl.BlockSpec((1, 128), lambda j: (0, 0)),
        out_shape=jax.ShapeDtypeStruct((1, D), jnp.float32),
    )(w0, w1, f_pad, W2, b2)


def kernel(feature, edge_index, W1, b1, W2, b2):
    pad = jnp.full((2, EPAD - E), SENT, jnp.int32)
    ei = jnp.concatenate([edge_index, pad], axis=1)

    w_part = _mega_kernel()(ei[0], ei[1])

    f_pad = jnp.pad(feature, ((0, NPAD - N), (0, 0)))
    out = _final(
        w_part[0].reshape(_ROWS, 1, 128),
        w_part[1].reshape(_ROWS, 1, 128),
        f_pad,
        W2,
        b2.reshape(1, D),
    )
    return out


# speedup vs baseline: 2.3270x; 2.2150x over previous
import functools

import jax
import jax.numpy as jnp
from jax import lax
from jax.experimental import pallas as pl
from jax.experimental.pallas import tpu as pltpu
from jax.experimental.pallas import tpu_sc as plsc

N = 10000
E = 320000
D = 128

NC = 2
NS = 16
L = 16
U = 8

C2 = 20480
HALF = C2 // NC
TAIL = E - (NS - 1) * C2
TAIL1 = TAIL - HALF

NPAD = 10240
SLC = NPAD // NS


@functools.cache
def _mesh():
    return plsc.VectorSubcoreMesh(
        core_axis_name="c", subcore_axis_name="s", num_cores=NC, num_subcores=NS
    )


def _zero_fill(ref, n):
    z = jnp.zeros((L,), jnp.float32)

    def body(o, _):
        for k in range(U):
            ref[pl.ds((o * U + k) * L, L)] = z
        return 0

    lax.fori_loop(0, n // (L * U), body, 0)


def _rsqrt16(x):
    i = plsc.bitcast(x, jnp.int32)
    i = jnp.int32(0x5F3759DF) - (i >> 1)
    y = plsc.bitcast(i, jnp.float32)
    for _ in range(3):
        y = y * (jnp.float32(1.5) - jnp.float32(0.5) * x * y * y)
    return y


def _mega_body(ei_hbm, out_hbm,
               idx_s, idx_d, loc, rbuf, nin_loc, nout_s, wstage,
               grid_s, grid_d, sh_nin):
    cid = lax.axis_index("c")
    sid = lax.axis_index("s")

    @pl.when(sid < NS - 1)
    def _():
        pltpu.sync_copy(ei_hbm.at[0, pl.ds(sid * C2, C2)], idx_s)
        pltpu.sync_copy(ei_hbm.at[1, pl.ds(sid * C2, C2)], idx_d)

    @pl.when(sid == NS - 1)
    def _():
        pltpu.sync_copy(ei_hbm.at[0, pl.ds((NS - 1) * C2, TAIL)],
                        idx_s.at[pl.ds(0, TAIL)])
        pltpu.sync_copy(ei_hbm.at[1, pl.ds((NS - 1) * C2, TAIL)],
                        idx_d.at[pl.ds(0, TAIL)])

    nblk = jnp.where(sid < NS - 1, C2 // (L * U), TAIL // (L * U))
    ones16 = jnp.full((L,), 1.0, jnp.float32)

    def hist(idx_ref, grid):
        _zero_fill(loc, NPAD)

        def upd(o, _):
            idxs = [idx_ref[pl.ds((o * U + k) * L, L)] for k in range(U)]
            for k in range(U):
                plsc.addupdate_scatter(loc, [idxs[k]], ones16)
            return 0

        lax.fori_loop(0, nblk, upd, 0)
        pltpu.sync_copy(loc, grid.at[sid])

    hist(idx_s, grid_s)
    hist(idx_d, grid_d)
    plsc.subcore_barrier()

    def reduce_norm(grid, dst_buf):
        pltpu.sync_copy(grid.at[:, pl.ds(sid * SLC, SLC)], rbuf)

        def red(i, _):
            t = [rbuf[r, pl.ds(i * L, L)] for r in range(NS)]
            while len(t) > 1:
                t = [a + b for a, b in zip(t[::2], t[1::2])]
            dst_buf[pl.ds(i * L, L)] = _rsqrt16(jnp.maximum(t[0], 1.0))
            return 0

        lax.fori_loop(0, SLC // L, red, 0)

    reduce_norm(grid_s, nout_s)
    reduce_norm(grid_d, wstage)
    pltpu.sync_copy(wstage, sh_nin.at[pl.ds(sid * SLC, SLC)])
    plsc.subcore_barrier()

    pltpu.sync_copy(sh_nin, nin_loc)

    _zero_fill(loc, NPAD)
    base = cid * HALF

    def upd2(o, _):
        d16s = [idx_d[pl.ds(base + (o * U + k) * L, L)] for k in range(U)]
        v16s = [plsc.load_gather(nin_loc, [d]) for d in d16s]
        s16s = [idx_s[pl.ds(base + (o * U + k) * L, L)] for k in range(U)]
        for k in range(U):
            plsc.addupdate_scatter(loc, [s16s[k]], v16s[k])
        return 0

    nblk2 = jnp.where(jnp.logical_or(sid < NS - 1, cid == 0),
                      HALF // (L * U), TAIL1 // (L * U))
    lax.fori_loop(0, nblk2, upd2, 0)
    pltpu.sync_copy(loc, grid_s.at[sid])
    plsc.subcore_barrier()

    pltpu.sync_copy(grid_s.at[:, pl.ds(sid * SLC, SLC)], rbuf)
    inv_n = jnp.float32(1.0 / N)

    def red2(i, _):
        t = [rbuf[r, pl.ds(i * L, L)] for r in range(NS)]
        while len(t) > 1:
            t = [a + b for a, b in zip(t[::2], t[1::2])]
        wstage[pl.ds(i * L, L)] = t[0] * nout_s[pl.ds(i * L, L)] * inv_n
        return 0

    lax.fori_loop(0, SLC // L, red2, 0)
    pltpu.sync_copy(wstage, out_hbm.at[cid, pl.ds(sid * SLC, SLC)])


@functools.cache
def _mega_kernel():
    return pl.kernel(
        _mega_body,
        out_type=jax.ShapeDtypeStruct((NC, NPAD), jnp.float32),
        mesh=_mesh(),
        compiler_params=pltpu.CompilerParams(needs_layout_passes=False),
        scratch_types=[
            pltpu.VMEM((C2,), jnp.int32),
            pltpu.VMEM((C2,), jnp.int32),
            pltpu.VMEM((NPAD,), jnp.float32),
            pltpu.VMEM((NS, SLC), jnp.float32),
            pltpu.VMEM((NPAD,), jnp.float32),
            pltpu.VMEM((SLC,), jnp.float32),
            pltpu.VMEM((SLC,), jnp.float32),
            pltpu.VMEM_SHARED((NS, NPAD), jnp.float32),
            pltpu.VMEM_SHARED((NS, NPAD), jnp.float32),
            pltpu.VMEM_SHARED((NPAD,), jnp.float32),
        ],
    )


_R = 1000
_NG = N // _R


def _final_body(w0_ref, w1_ref, f_ref, w2_ref, b2_ref, out_ref):
    j = pl.program_id(0)
    w = w0_ref[...] + w1_ref[...]
    part = jnp.sum(f_ref[...] * w, axis=0, keepdims=True)

    @pl.when(j == 0)
    def _():
        out_ref[...] = jnp.zeros_like(out_ref)

    out_ref[...] += part

    @pl.when(j == _NG - 1)
    def _():
        out_ref[...] = (
            jnp.dot(out_ref[...], w2_ref[...], preferred_element_type=jnp.float32)
            + b2_ref[...]
        )


def _final(w0, w1, feature, W2, b2):
    return pl.pallas_call(
        _final_body,
        grid=(_NG,),
        in_specs=[
            pl.BlockSpec((_R, 1), lambda j: (j, 0)),
            pl.BlockSpec((_R, 1), lambda j: (j, 0)),
            pl.BlockSpec((_R, 128), lambda j: (j, 0)),
            pl.BlockSpec((128, 128), lambda j: (0, 0)),
            pl.BlockSpec((1, 128), lambda j: (0, 0)),
        ],
        out_specs=pl.BlockSpec((1, 128), lambda j: (0, 0)),
        out_shape=jax.ShapeDtypeStruct((1, D), jnp.float32),
    )(w0, w1, feature, W2, b2)


def kernel(feature, edge_index, W1, b1, W2, b2):
    w_part = _mega_kernel()(edge_index)
    out = _final(
        w_part[0, :N].reshape(N, 1),
        w_part[1, :N].reshape(N, 1),
        feature,
        W2,
        b2.reshape(1, D),
    )
    return out


# speedup vs baseline: 2.3428x; 1.0068x over previous
import functools

import jax
import jax.numpy as jnp
from jax import lax
from jax.experimental import pallas as pl
from jax.experimental.pallas import tpu as pltpu
from jax.experimental.pallas import tpu_sc as plsc

N = 10000
E = 320000
D = 128

NC = 2
NS = 16
L = 16
U = 8

C2 = 20480
HALF = C2 // NC
TAIL = E - (NS - 1) * C2
TAIL1 = TAIL - HALF

NPAD = 10240
SLC = NPAD // NS


@functools.cache
def _mesh():
    return plsc.VectorSubcoreMesh(
        core_axis_name="c", subcore_axis_name="s", num_cores=NC, num_subcores=NS
    )


def _zero_fill(ref, n):
    z = jnp.zeros((L,), jnp.float32)
    zu = 16

    def body(o, _):
        for k in range(zu):
            ref[pl.ds((o * zu + k) * L, L)] = z
        return 0

    lax.fori_loop(0, n // (L * zu), body, 0)


def _rsqrt16(x):
    i = plsc.bitcast(x, jnp.int32)
    i = jnp.int32(0x5F3759DF) - (i >> 1)
    y = plsc.bitcast(i, jnp.float32)
    for _ in range(3):
        y = y * (jnp.float32(1.5) - jnp.float32(0.5) * x * y * y)
    return y


def _mega_body(ei_hbm, out_hbm,
               idx_s, idx_d, loc, loc_d, rbuf, nin_loc, nout_s, wstage,
               grid_s, grid_d, sh_nin):
    cid = lax.axis_index("c")
    sid = lax.axis_index("s")

    @pl.when(sid < NS - 1)
    def _():
        pltpu.sync_copy(ei_hbm.at[0, pl.ds(sid * C2, C2)], idx_s)
        pltpu.sync_copy(ei_hbm.at[1, pl.ds(sid * C2, C2)], idx_d)

    @pl.when(sid == NS - 1)
    def _():
        pltpu.sync_copy(ei_hbm.at[0, pl.ds((NS - 1) * C2, TAIL)],
                        idx_s.at[pl.ds(0, TAIL)])
        pltpu.sync_copy(ei_hbm.at[1, pl.ds((NS - 1) * C2, TAIL)],
                        idx_d.at[pl.ds(0, TAIL)])

    nblk = jnp.where(sid < NS - 1, C2 // (L * U), TAIL // (L * U))
    ones16 = jnp.full((L,), 1.0, jnp.float32)

    _zero_fill(loc, NPAD)
    _zero_fill(loc_d, NPAD)

    def upd(o, _):
        s_idxs = [idx_s[pl.ds((o * U + k) * L, L)] for k in range(U)]
        d_idxs = [idx_d[pl.ds((o * U + k) * L, L)] for k in range(U)]
        for k in range(U):
            plsc.addupdate_scatter(loc, [s_idxs[k]], ones16)
            plsc.addupdate_scatter(loc_d, [d_idxs[k]], ones16)
        return 0

    lax.fori_loop(0, nblk, upd, 0)
    pltpu.sync_copy(loc, grid_s.at[sid])
    pltpu.sync_copy(loc_d, grid_d.at[sid])
    plsc.subcore_barrier()

    def reduce_norm(grid, dst_buf):
        pltpu.sync_copy(grid.at[:, pl.ds(sid * SLC, SLC)], rbuf)

        def red(i, _):
            t = [rbuf[r, pl.ds(i * L, L)] for r in range(NS)]
            while len(t) > 1:
                t = [a + b for a, b in zip(t[::2], t[1::2])]
            dst_buf[pl.ds(i * L, L)] = _rsqrt16(jnp.maximum(t[0], 1.0))
            return 0

        lax.fori_loop(0, SLC // L, red, 0)

    reduce_norm(grid_s, nout_s)
    reduce_norm(grid_d, wstage)
    pltpu.sync_copy(wstage, sh_nin.at[pl.ds(sid * SLC, SLC)])
    plsc.subcore_barrier()

    pltpu.sync_copy(sh_nin, nin_loc)

    _zero_fill(loc, NPAD)
    base = cid * HALF

    def upd2(o, _):
        d16s = [idx_d[pl.ds(base + (o * U + k) * L, L)] for k in range(U)]
        v16s = [plsc.load_gather(nin_loc, [d]) for d in d16s]
        s16s = [idx_s[pl.ds(base + (o * U + k) * L, L)] for k in range(U)]
        for k in range(U):
            plsc.addupdate_scatter(loc, [s16s[k]], v16s[k])
        return 0

    nblk2 = jnp.where(jnp.logical_or(sid < NS - 1, cid == 0),
                      HALF // (L * U), TAIL1 // (L * U))
    lax.fori_loop(0, nblk2, upd2, 0)
    pltpu.sync_copy(loc, grid_s.at[sid])
    plsc.subcore_barrier()

    pltpu.sync_copy(grid_s.at[:, pl.ds(sid * SLC, SLC)], rbuf)
    inv_n = jnp.float32(1.0 / N)

    def red2(i, _):
        t = [rbuf[r, pl.ds(i * L, L)] for r in range(NS)]
        while len(t) > 1:
            t = [a + b for a, b in zip(t[::2], t[1::2])]
        wstage[pl.ds(i * L, L)] = t[0] * nout_s[pl.ds(i * L, L)] * inv_n
        return 0

    lax.fori_loop(0, SLC // L, red2, 0)
    pltpu.sync_copy(wstage, out_hbm.at[cid, pl.ds(sid * SLC, SLC)])


@functools.cache
def _mega_kernel():
    return pl.kernel(
        _mega_body,
        out_type=jax.ShapeDtypeStruct((NC, NPAD), jnp.float32),
        mesh=_mesh(),
        compiler_params=pltpu.CompilerParams(needs_layout_passes=False),
        scratch_types=[
            pltpu.VMEM((C2,), jnp.int32),
            pltpu.VMEM((C2,), jnp.int32),
            pltpu.VMEM((NPAD,), jnp.float32),
            pltpu.VMEM((NPAD,), jnp.float32),
            pltpu.VMEM((NS, SLC), jnp.float32),
            pltpu.VMEM((NPAD,), jnp.float32),
            pltpu.VMEM((SLC,), jnp.float32),
            pltpu.VMEM((SLC,), jnp.float32),
            pltpu.VMEM_SHARED((NS, NPAD), jnp.float32),
            pltpu.VMEM_SHARED((NS, NPAD), jnp.float32),
            pltpu.VMEM_SHARED((NPAD,), jnp.float32),
        ],
    )


_R = 1000
_NG = N // _R


def _final_body(wp_ref, f_ref, w2_ref, b2_ref, out_ref):
    j = pl.program_id(0)
    w = wp_ref[0] + wp_ref[1]
    part = jnp.sum(f_ref[...] * w, axis=0, keepdims=True)

    @pl.when(j == 0)
    def _():
        out_ref[...] = jnp.zeros_like(out_ref)

    out_ref[...] += part

    @pl.when(j == _NG - 1)
    def _():
        out_ref[...] = (
            jnp.dot(out_ref[...], w2_ref[...], preferred_element_type=jnp.float32)
            + b2_ref[...]
        )


def _final(wp, feature, W2, b2):
    return pl.pallas_call(
        _final_body,
        grid=(_NG,),
        in_specs=[
            pl.BlockSpec((NC, _R, 1), lambda j: (0, j, 0)),
            pl.BlockSpec((_R, 128), lambda j: (j, 0)),
            pl.BlockSpec((128, 128), lambda j: (0, 0)),
            pl.BlockSpec((1, 128), lambda j: (0, 0)),
        ],
        out_specs=pl.BlockSpec((1, 128), lambda j: (0, 0)),
        out_shape=jax.ShapeDtypeStruct((1, D), jnp.float32),
    )(wp, feature, W2, b2)


def kernel(feature, edge_index, W1, b1, W2, b2):
    w_part = _mega_kernel()(edge_index)
    out = _final(w_part.reshape(NC, NPAD, 1), feature, W2, b2.reshape(1, D))
    return out
